# Initial kernel scaffold; baseline (speedup 1.0000x reference)
import jax
import jax.numpy as jnp
from jax.experimental import pallas as pl


def kernel(boxes, x1_w, y1_w, x2_w, y2_w, w_w, h_w, cx_w, cy_w, label_w):
    raise NotImplementedError("write your pallas kernel here")


# speedup vs baseline: 1.6753x; 1.6753x over previous
import jax
import jax.numpy as jnp
from jax import lax
from jax.experimental import pallas as pl
from jax.experimental.pallas import tpu as pltpu
from jax.experimental.pallas import tpu_sc as plsc

BBOX_SIZE = 1000
HIDDEN = 512
NTAB = 9
NC, NS = 2, 16
NW = NC * NS
BATCH, NBOX = 1024, 200
R = BATCH * NBOX
C = 128
S = 16
RPW = R // NW
NCHUNK = RPW // C
NSUB = C // S
LG = HIDDEN // 16


def _sc_body(table, boxes_t, out, f_v, idx_v, buf_v, stage_v, sem):
    wid = lax.axis_index("s") * NC + lax.axis_index("c")
    w_base = wid * RPW

    @pl.loop(0, NCHUNK)
    def _chunk(ci):
        base = w_base + ci * C
        pltpu.sync_copy(boxes_t.at[:, pl.ds(base, C)], f_v)
        for g in range(C // 16):
            s = pl.ds(g * 16, 16)
            cx = f_v[0, s]
            cy = f_v[1, s]
            w = f_v[2, s]
            h = f_v[3, s]
            lab = f_v[4, s]
            hw = lax.shift_right_arithmetic(w, 1)
            hh = lax.shift_right_arithmetic(h, 1)
            zero = jnp.int32(0)
            top = jnp.int32(BBOX_SIZE - 1)
            x1 = jnp.minimum(jnp.maximum(cx - hw, zero), top)
            y1 = jnp.minimum(jnp.maximum(cy - hh, zero), top)
            x2 = jnp.minimum(jnp.maximum(cx + hw, zero), top)
            y2 = jnp.minimum(jnp.maximum(cy + hh, zero), top)
            vals = (x1, y1 + 1000, x2 + 2000, y2 + 3000, w + 4000,
                    h + 5000, cx + 6000, cy + 7000, lab + 8000)
            for j, v in enumerate(vals):
                idx_v[j, s] = v

        @pl.loop(0, NSUB)
        def _sub(si):
            cps = [
                pltpu.async_copy(
                    table.at[idx_v.at[j, pl.ds(si * S, S)]],
                    buf_v.at[pl.ds(j * S, S)], sem,
                )
                for j in range(NTAB)
            ]
            for cp in cps:
                cp.wait()

            @pl.loop(0, S)
            def _row(r):
                for l in range(LG):
                    ls = pl.ds(l * 16, 16)
                    a = buf_v[r, ls]
                    for j in range(1, NTAB):
                        a = a + buf_v[j * S + r, ls]
                    stage_v[r, ls] = a

            pltpu.sync_copy(stage_v, out.at[pl.ds(base + si * S, S)])


def kernel(boxes, x1_w, y1_w, x2_w, y2_w, w_w, h_w, cx_w, cy_w, label_w):
    table = jnp.concatenate(
        [x1_w, y1_w, x2_w, y2_w, w_w, h_w, cx_w, cy_w, label_w], axis=0
    )
    boxes_t = boxes.reshape(R, 5).T

    run = pl.kernel(
        _sc_body,
        out_type=jax.ShapeDtypeStruct((R, HIDDEN), jnp.float32),
        mesh=plsc.VectorSubcoreMesh(
            core_axis_name="c", subcore_axis_name="s", num_cores=NC, num_subcores=NS
        ),
        scratch_types=[
            pltpu.VMEM((5, C), jnp.int32),
            pltpu.VMEM((NTAB, C), jnp.int32),
            pltpu.VMEM((NTAB * S, HIDDEN), jnp.float32),
            pltpu.VMEM((S, HIDDEN), jnp.float32),
            pltpu.SemaphoreType.DMA,
        ],
    )
    out = run(table, boxes_t)
    return out.reshape(BATCH, NBOX, HIDDEN)


# speedup vs baseline: 1.7007x; 1.0151x over previous
import jax
import jax.numpy as jnp
from jax import lax
from jax.experimental import pallas as pl
from jax.experimental.pallas import tpu as pltpu
from jax.experimental.pallas import tpu_sc as plsc

BBOX_SIZE = 1000
HIDDEN = 512
NTAB = 9
NC, NS = 2, 16
NW = NC * NS
BATCH, NBOX = 1024, 200
R = BATCH * NBOX
C = 128
S = 8
G = S * NTAB
RPW = R // NW
NCHUNK = RPW // C
NSUB = C // S
LG = HIDDEN // 16


def _sc_body(table, boxes_t, out, f_v, idx_v,
             buf0, buf1, st0, st1, gsem0, gsem1, ssem0, ssem1):
    wid = lax.axis_index("s") * NC + lax.axis_index("c")
    w_base = wid * RPW
    bufs = (buf0, buf1)
    stages = (st0, st1)
    gsems = (gsem0, gsem1)
    ssems = (ssem0, ssem1)

    def fire(si, par):
        for j in range(NTAB):
            pltpu.async_copy(
                table.at[idx_v.at[j, pl.ds(si * S, S)]],
                bufs[par].at[pl.ds(j * S, S)], gsems[par],
            )

    def wait_gather(par):
        for j in range(NTAB):
            pltpu.make_async_copy(
                table.at[idx_v.at[j, pl.ds(0, S)]],
                bufs[par].at[pl.ds(j * S, S)], gsems[par],
            ).wait()

    def accumulate(par):
        buf = bufs[par]
        stage = stages[par]

        @pl.loop(0, S)
        def _row(r):
            for l in range(LG):
                ls = pl.ds(l * 16, 16)
                a0 = buf[r, ls] + buf[S + r, ls]
                a1 = buf[2 * S + r, ls] + buf[3 * S + r, ls]
                a2 = buf[4 * S + r, ls] + buf[5 * S + r, ls]
                a3 = buf[6 * S + r, ls] + buf[7 * S + r, ls]
                stage[r, ls] = ((a0 + a1) + (a2 + a3)) + buf[8 * S + r, ls]

    def fire_store(base, si, par):
        pltpu.async_copy(
            stages[par], out.at[pl.ds(base + si * S, S)], ssems[par]
        )

    def wait_store(par):
        pltpu.make_async_copy(
            stages[par], out.at[pl.ds(0, S)], ssems[par]
        ).wait()

    @pl.loop(0, NCHUNK)
    def _chunk(ci):
        base = w_base + ci * C
        pltpu.sync_copy(boxes_t.at[:, pl.ds(base, C)], f_v)
        for g in range(C // 16):
            s = pl.ds(g * 16, 16)
            cx = f_v[0, s]
            cy = f_v[1, s]
            w = f_v[2, s]
            h = f_v[3, s]
            lab = f_v[4, s]
            hw = lax.shift_right_arithmetic(w, 1)
            hh = lax.shift_right_arithmetic(h, 1)
            zero = jnp.int32(0)
            top = jnp.int32(BBOX_SIZE - 1)
            x1 = jnp.minimum(jnp.maximum(cx - hw, zero), top)
            y1 = jnp.minimum(jnp.maximum(cy - hh, zero), top)
            x2 = jnp.minimum(jnp.maximum(cx + hw, zero), top)
            y2 = jnp.minimum(jnp.maximum(cy + hh, zero), top)
            vals = (x1, y1 + 1000, x2 + 2000, y2 + 3000, w + 4000,
                    h + 5000, cx + 6000, cy + 7000, lab + 8000)
            for j, v in enumerate(vals):
                idx_v[j, s] = v
        fire(0, 0)
        @pl.loop(0, NSUB // 2)
        def _pair(pi):
            s0 = pi * 2
            for par in (0, 1):
                si = s0 + par
                wait_gather(par)
                @pl.when(si + 1 < NSUB)
                def _():
                    fire(si + 1, 1 - par)
                @pl.when(jnp.logical_or(ci > 0, pi > 0))
                def _():
                    wait_store(par)
                accumulate(par)
                fire_store(base, si, par)


def kernel(boxes, x1_w, y1_w, x2_w, y2_w, w_w, h_w, cx_w, cy_w, label_w):
    table = jnp.concatenate(
        [x1_w, y1_w, x2_w, y2_w, w_w, h_w, cx_w, cy_w, label_w], axis=0
    )
    boxes_t = boxes.reshape(R, 5).T

    run = pl.kernel(
        _sc_body,
        out_type=jax.ShapeDtypeStruct((R, HIDDEN), jnp.float32),
        mesh=plsc.VectorSubcoreMesh(
            core_axis_name="c", subcore_axis_name="s", num_cores=NC, num_subcores=NS
        ),
        scratch_types=[
            pltpu.VMEM((5, C), jnp.int32),
            pltpu.VMEM((NTAB, C), jnp.int32),
            pltpu.VMEM((G, HIDDEN), jnp.float32),
            pltpu.VMEM((G, HIDDEN), jnp.float32),
            pltpu.VMEM((S, HIDDEN), jnp.float32),
            pltpu.VMEM((S, HIDDEN), jnp.float32),
            pltpu.SemaphoreType.DMA,
            pltpu.SemaphoreType.DMA,
            pltpu.SemaphoreType.DMA,
            pltpu.SemaphoreType.DMA,
        ],
    )
    out = run(table, boxes_t)
    return out.reshape(BATCH, NBOX, HIDDEN)


# speedup vs baseline: 5.2478x; 3.0857x over previous
import jax
import jax.numpy as jnp
from jax import lax
from jax.experimental import pallas as pl
from jax.experimental.pallas import tpu as pltpu
from jax.experimental.pallas import tpu_sc as plsc

BBOX_SIZE = 1000
HIDDEN = 512
NTAB = 9
VOCAB = NTAB * 1000
NC, NS = 2, 16
NW = NC * NS
BATCH, NBOX = 1024, 200
R = BATCH * NBOX
C = 128
S = 16
NQ = 4
Q = HIDDEN // NQ
RPW = R // NW
NCHUNK = RPW // C
NSUB = C // S
LQ = Q // 16
NLOAD = 9
LROWS = VOCAB // NLOAD


def _sc_body(tq0, tq1, tq2, tq3, boxes_t, out,
             f_v, idx_v, buf0, buf1, st0, st1, tab_sh,
             gsem0, gsem1, ssem0, ssem1, lsem):
    cid = lax.axis_index("c")
    sid = lax.axis_index("s")
    wid = sid * NC + cid
    w_base = wid * RPW
    bufs = (buf0, buf1)
    stages = (st0, st1)
    gsems = (gsem0, gsem1)
    ssems = (ssem0, ssem1)

    def fire(si, par):
        for j in range(NTAB):
            pltpu.async_copy(
                tab_sh.at[idx_v.at[j, pl.ds(si * S, S)]],
                bufs[par].at[pl.ds(j * S, S)], gsems[par],
            )

    def wait_gather(par):
        for j in range(NTAB):
            pltpu.make_async_copy(
                tab_sh.at[idx_v.at[j, pl.ds(0, S)]],
                bufs[par].at[pl.ds(j * S, S)], gsems[par],
            ).wait()

    def accumulate(par):
        buf = bufs[par]
        stage = stages[par]

        @pl.loop(0, S)
        def _row(r):
            for l in range(LQ):
                ls = pl.ds(l * 16, 16)
                a0 = buf[r, ls] + buf[S + r, ls]
                a1 = buf[2 * S + r, ls] + buf[3 * S + r, ls]
                a2 = buf[4 * S + r, ls] + buf[5 * S + r, ls]
                a3 = buf[6 * S + r, ls] + buf[7 * S + r, ls]
                stage[r, ls] = ((a0 + a1) + (a2 + a3)) + buf[8 * S + r, ls]

    for q, tq in enumerate((tq0, tq1, tq2, tq3)):
        qcol = q * Q

        def fire_store(base, si, par, qcol=qcol):
            pltpu.async_copy(
                stages[par],
                out.at[pl.ds(base + si * S, S), pl.ds(qcol, Q)], ssems[par],
            )

        def wait_store(par, qcol=qcol):
            pltpu.make_async_copy(
                stages[par], out.at[pl.ds(0, S), pl.ds(qcol, Q)], ssems[par]
            ).wait()

        @pl.when(sid < NLOAD)
        def _load():
            pltpu.async_copy(
                tq.at[pl.ds(sid * LROWS, LROWS)],
                tab_sh.at[pl.ds(sid * LROWS, LROWS)], lsem,
            ).wait()
        plsc.subcore_barrier()

        @pl.loop(0, NCHUNK)
        def _chunk(ci):
            base = w_base + ci * C
            pltpu.sync_copy(boxes_t.at[:, pl.ds(base, C)], f_v)
            for g in range(C // 16):
                s = pl.ds(g * 16, 16)
                cx = f_v[0, s]
                cy = f_v[1, s]
                w = f_v[2, s]
                h = f_v[3, s]
                lab = f_v[4, s]
                hw = lax.shift_right_arithmetic(w, 1)
                hh = lax.shift_right_arithmetic(h, 1)
                zero = jnp.int32(0)
                top = jnp.int32(BBOX_SIZE - 1)
                x1 = jnp.minimum(jnp.maximum(cx - hw, zero), top)
                y1 = jnp.minimum(jnp.maximum(cy - hh, zero), top)
                x2 = jnp.minimum(jnp.maximum(cx + hw, zero), top)
                y2 = jnp.minimum(jnp.maximum(cy + hh, zero), top)
                vals = (x1, y1 + 1000, x2 + 2000, y2 + 3000, w + 4000,
                        h + 5000, cx + 6000, cy + 7000, lab + 8000)
                for j, v in enumerate(vals):
                    idx_v[j, s] = v

            fire(0, 0)

            @pl.loop(0, NSUB // 2)
            def _pair(pi):
                s0 = pi * 2
                for par in (0, 1):
                    si = s0 + par
                    wait_gather(par)

                    @pl.when(si + 1 < NSUB)
                    def _():
                        fire(si + 1, 1 - par)

                    @pl.when(jnp.logical_or(ci > 0, pi > 0))
                    def _():
                        wait_store(par)
                    accumulate(par)
                    fire_store(base, si, par)

        wait_store(0)
        wait_store(1)
        plsc.subcore_barrier()


def kernel(boxes, x1_w, y1_w, x2_w, y2_w, w_w, h_w, cx_w, cy_w, label_w):
    table = jnp.concatenate(
        [x1_w, y1_w, x2_w, y2_w, w_w, h_w, cx_w, cy_w, label_w], axis=0
    )
    tq = table.reshape(VOCAB, NQ, Q).transpose(1, 0, 2)
    boxes_t = boxes.reshape(R, 5).T

    run = pl.kernel(
        _sc_body,
        out_type=jax.ShapeDtypeStruct((R, HIDDEN), jnp.float32),
        mesh=plsc.VectorSubcoreMesh(
            core_axis_name="c", subcore_axis_name="s", num_cores=NC, num_subcores=NS
        ),
        scratch_types=[
            pltpu.VMEM((5, C), jnp.int32),
            pltpu.VMEM((NTAB, C), jnp.int32),
            pltpu.VMEM((NTAB * S, Q), jnp.float32),
            pltpu.VMEM((NTAB * S, Q), jnp.float32),
            pltpu.VMEM((S, Q), jnp.float32),
            pltpu.VMEM((S, Q), jnp.float32),
            pltpu.VMEM_SHARED((VOCAB, Q), jnp.float32),
            pltpu.SemaphoreType.DMA,
            pltpu.SemaphoreType.DMA,
            pltpu.SemaphoreType.DMA,
            pltpu.SemaphoreType.DMA,
            pltpu.SemaphoreType.DMA,
        ],
    )
    out = run(tq[0], tq[1], tq[2], tq[3], boxes_t)
    return out.reshape(BATCH, NBOX, HIDDEN)
